# baseline (speedup 1.0000x reference)
import dataclasses
import functools

import jax
import jax.numpy as jnp
from jax import lax
from jax.experimental import pallas as pl
from jax.experimental.pallas import tpu as pltpu
from jax.experimental.pallas import tpu_sc as plsc

N = 10000
E = 320000
D = 128

NC = 2
NS = 16
L = 16
NW = NC * NS

CH = 80
NB = 4
GA = 2
CPW = E // (NW * CH)
BCH = 4
NBAT = 4
CPW_PAD = 128
SLOTS = BCH * NBAT
ROUNDS = (CPW + SLOTS - 1) // SLOTS
ZCH = 16
ZROWS = 624
ZPT = (N - (NS - 1) * ZROWS) // ZCH

_SPLAT_DNUMS = lax.GatherDimensionNumbers(
    offset_dims=(), collapsed_slice_dims=(0,), start_index_map=(0,))


def _splat(vec, lane):
    idx = jnp.full((L, 1), lane, jnp.int32)
    return lax.gather(vec, idx, _SPLAT_DNUMS, (1,),
                      mode=lax.GatherScatterMode.PROMISE_IN_BOUNDS)


def _sc_agg_body(sup_hbm, crw_hbm, z_hbm, out_hbm, acc, *rest):
    cbigs = rest[:NBAT]
    rbufs = rest[NBAT:NBAT + NB]
    csems = rest[NBAT + NB:2 * NBAT + NB]
    gsems = rest[2 * NBAT + NB:2 * NBAT + 2 * NB]
    ssems = rest[2 * NBAT + 2 * NB:]
    cid = lax.axis_index("c")
    sid = lax.axis_index("s")
    wid = cid * NS + sid

    rbase = sid * ZROWS
    ntr = jnp.where(sid == NS - 1, ZPT, ZROWS // ZCH)

    @pl.loop(0, ZPT)
    def _(k):
        @pl.when(k < ntr)
        def _():
            pltpu.sync_copy(z_hbm, acc.at[pl.ds(rbase + k * ZCH, ZCH)])

    plsc.subcore_barrier()

    def cslice(pos):
        return cbigs[(pos % SLOTS) // BCH].at[pos % BCH]

    for m in range(NBAT - 1):
        pltpu.async_copy(crw_hbm.at[wid, pl.ds(m * BCH, BCH)], cbigs[m],
                         csems[m])
    pltpu.make_async_copy(crw_hbm.at[wid, pl.ds(0, BCH)], cbigs[0],
                          csems[0]).wait()
    for c in range(GA):
        pltpu.async_copy(sup_hbm.at[cslice(c).at[0]], rbufs[c], gsems[c])

    @pl.loop(0, ROUNDS)
    def _(p):
        i0 = p * SLOTS
        for b in range(SLOTS):
            i = i0 + b
            rb = b % NB
            nrb = (b + GA) % NB

            @pl.when(jnp.logical_and(GA <= i, i < CPW + GA))
            def _():
                pltpu.make_async_copy(
                    rbufs[nrb], acc.at[cslice(b - GA).at[1]],
                    ssems[nrb]).wait()

            if b % BCH == GA:
                mb = i // BCH + NBAT - 1
                fb = (b // BCH + NBAT - 1) % NBAT

                @pl.when(mb * BCH < CPW)
                def _():
                    pltpu.async_copy(crw_hbm.at[wid, pl.ds(mb * BCH, BCH)],
                                     cbigs[fb], csems[fb])

            @pl.when(i + GA < CPW)
            def _():
                if (b + GA) % BCH == 0:
                    nmb = ((b + GA) % SLOTS) // BCH
                    pltpu.make_async_copy(
                        crw_hbm.at[wid, pl.ds((i + GA) // BCH * BCH, BCH)],
                        cbigs[nmb], csems[nmb]).wait()
                pltpu.async_copy(sup_hbm.at[cslice(b + GA).at[0]],
                                 rbufs[nrb], gsems[nrb])

            @pl.when(i < CPW)
            def _():
                pltpu.make_async_copy(sup_hbm.at[cslice(b).at[0]],
                                      rbufs[rb], gsems[rb]).wait()

                rbuf = rbufs[rb]
                cbuf = cslice(b)

                @pl.loop(0, CH // L)
                def _(g):
                    wvec = plsc.bitcast(cbuf[2, pl.ds(g * L, L)], jnp.float32)
                    for l in range(L):
                        sp = _splat(wvec, l)
                        e = g * L + l
                        for q in range(D // L):
                            sl = pl.ds(q * L, L)
                            rbuf[e, sl] = rbuf[e, sl] * sp

                pltpu.async_copy(rbuf, acc.at[cbuf.at[1]], ssems[rb],
                                 add=True)

    plsc.subcore_barrier()

    @pl.loop(0, ZPT)
    def _(k):
        @pl.when(k < ntr)
        def _():
            off = rbase + k * ZCH
            pltpu.sync_copy(acc.at[pl.ds(off, ZCH)],
                            out_hbm.at[cid, pl.ds(off, ZCH)])


@functools.cache
def _get_sc_agg():
    mesh = plsc.VectorSubcoreMesh(core_axis_name="c", subcore_axis_name="s",
                                  num_cores=NC, num_subcores=NS)
    cp = pltpu.CompilerParams()
    if "needs_layout_passes" in pltpu.CompilerParams.__dataclass_fields__:
        cp = dataclasses.replace(cp, needs_layout_passes=False)
    return pl.kernel(
        _sc_agg_body,
        out_type=jax.ShapeDtypeStruct((NC, N, D), jnp.float32),
        mesh=mesh,
        compiler_params=cp,
        scratch_types=[
            pltpu.VMEM_SHARED((N, D), jnp.float32),
        ]
        + [pltpu.VMEM((BCH, 3, CH), jnp.int32) for _ in range(NBAT)]
        + [pltpu.VMEM((CH, D), jnp.float32) for _ in range(NB)]
        + [pltpu.SemaphoreType.DMA for _ in range(NBAT)]
        + [pltpu.SemaphoreType.DMA for _ in range(2 * NB)],
    )


BLK = 2000


def _mm1_body(x_ref, w_ref, o_ref):
    o_ref[...] = jnp.dot(x_ref[...], w_ref[...],
                         preferred_element_type=jnp.float32,
                         precision=lax.Precision.DEFAULT)


def _mid_body(p_ref, b_ref, w_ref, o_ref):
    h = p_ref[0] + p_ref[1] + b_ref[...]
    h = jnp.maximum(h, 0.0)
    o_ref[...] = jnp.dot(h, w_ref[...],
                         preferred_element_type=jnp.float32,
                         precision=lax.Precision.DEFAULT)


def _fin_body(p_ref, b_ref, o_ref):
    o_ref[...] = p_ref[0] + p_ref[1] + b_ref[...]


_mm1 = pl.pallas_call(
    _mm1_body,
    grid=(N // BLK,),
    in_specs=[
        pl.BlockSpec((BLK, D), lambda i: (i, 0)),
        pl.BlockSpec((D, D), lambda i: (0, 0)),
    ],
    out_specs=pl.BlockSpec((BLK, D), lambda i: (i, 0)),
    out_shape=jax.ShapeDtypeStruct((N, D), jnp.float32),
)

_mid = pl.pallas_call(
    _mid_body,
    grid=(N // BLK,),
    in_specs=[
        pl.BlockSpec((NC, BLK, D), lambda i: (0, i, 0)),
        pl.BlockSpec((1, D), lambda i: (0, 0)),
        pl.BlockSpec((D, D), lambda i: (0, 0)),
    ],
    out_specs=pl.BlockSpec((BLK, D), lambda i: (i, 0)),
    out_shape=jax.ShapeDtypeStruct((N, D), jnp.float32),
)

_fin = pl.pallas_call(
    _fin_body,
    grid=(N // BLK,),
    in_specs=[
        pl.BlockSpec((NC, BLK, D), lambda i: (0, i, 0)),
        pl.BlockSpec((1, D), lambda i: (0, 0)),
    ],
    out_specs=pl.BlockSpec((BLK, D), lambda i: (i, 0)),
    out_shape=jax.ShapeDtypeStruct((N, D), jnp.float32),
)


@jax.jit
def kernel(x, edge_index, edge_weight, W1, b1, W2, b2):
    ei = edge_index.astype(jnp.int32)
    col3 = ei[1].reshape(NW, CPW, 1, CH)
    row3 = ei[0].reshape(NW, CPW, 1, CH)
    wbits = lax.bitcast_convert_type(edge_weight, jnp.int32)
    w3 = wbits.reshape(NW, CPW, 1, CH)
    crw = jnp.concatenate([col3, row3, w3], axis=2)
    crw = jnp.pad(crw, ((0, 0), (0, CPW_PAD - CPW), (0, 0), (0, 0)))
    b1r = b1.reshape(1, D)
    b2r = b2.reshape(1, D)

    zs = jnp.zeros((ZCH, D), jnp.float32)
    sc_agg = _get_sc_agg()
    s1 = _mm1(x, W1)
    p = sc_agg(s1, crw, zs)
    s2 = _mid(p, b1r, W2)
    q = sc_agg(s2, crw, zs)
    return _fin(q, b2r)


# speedup vs baseline: 1.2304x; 1.2304x over previous
import dataclasses
import functools

import jax
import jax.numpy as jnp
from jax import lax
from jax.experimental import pallas as pl
from jax.experimental.pallas import tpu as pltpu
from jax.experimental.pallas import tpu_sc as plsc

N = 10000
E = 320000
D = 128

NC = 2
NS = 16
L = 16
NW = NC * NS

CH = 80
NB = 4
GA = 2
CPW = E // (NW * CH)
BCH = 4
NBAT = 4
CPW_PAD = 128
SLOTS = BCH * NBAT
ROUNDS = (CPW + SLOTS - 1) // SLOTS
ZCH = 16
ZROWS = 624
ZPT = (N - (NS - 1) * ZROWS) // ZCH

_SPLAT_DNUMS = lax.GatherDimensionNumbers(
    offset_dims=(), collapsed_slice_dims=(0,), start_index_map=(0,))


def _splat(vec, lane):
    idx = jnp.full((L, 1), lane, jnp.int32)
    return lax.gather(vec, idx, _SPLAT_DNUMS, (1,),
                      mode=lax.GatherScatterMode.PROMISE_IN_BOUNDS)


def _sc_agg_body(sup_hbm, crw_hbm, out_hbm, acc, zbuf, *rest):
    cbigs = rest[:NBAT]
    rbufs = rest[NBAT:NBAT + NB]
    csems = rest[NBAT + NB:2 * NBAT + NB]
    gsems = rest[2 * NBAT + NB:2 * NBAT + 2 * NB]
    ssems = rest[2 * NBAT + 2 * NB:]
    cid = lax.axis_index("c")
    sid = lax.axis_index("s")
    wid = cid * NS + sid

    rbase = sid * ZROWS
    ntr = jnp.where(sid == NS - 1, ZPT, ZROWS // ZCH)

    zero = jnp.zeros((L,), jnp.float32)
    for r in range(ZCH // 2):
        for q in range(D // L):
            zbuf[r, pl.ds(q * L, L)] = zero

    @pl.loop(0, 2 * ZPT)
    def _(k):
        @pl.when(k < 2 * ntr)
        def _():
            pltpu.sync_copy(zbuf,
                            acc.at[pl.ds(rbase + k * (ZCH // 2), ZCH // 2)])

    plsc.subcore_barrier()

    def cslice(pos):
        return cbigs[(pos % SLOTS) // BCH].at[pos % BCH]

    for m in range(NBAT - 1):
        pltpu.async_copy(crw_hbm.at[wid, pl.ds(m * BCH, BCH)], cbigs[m],
                         csems[m])
    pltpu.make_async_copy(crw_hbm.at[wid, pl.ds(0, BCH)], cbigs[0],
                          csems[0]).wait()
    for c in range(GA):
        pltpu.async_copy(sup_hbm.at[cslice(c).at[0]], rbufs[c], gsems[c])

    @pl.loop(0, ROUNDS)
    def _(p):
        i0 = p * SLOTS
        for b in range(SLOTS):
            i = i0 + b
            rb = b % NB
            nrb = (b + GA) % NB

            @pl.when(jnp.logical_and(GA <= i, i < CPW + GA))
            def _():
                pltpu.make_async_copy(
                    rbufs[nrb], acc.at[cslice(b - GA).at[1]],
                    ssems[nrb]).wait()

            if b % BCH == GA:
                mb = i // BCH + NBAT - 1
                fb = (b // BCH + NBAT - 1) % NBAT

                @pl.when(mb * BCH < CPW)
                def _():
                    pltpu.async_copy(crw_hbm.at[wid, pl.ds(mb * BCH, BCH)],
                                     cbigs[fb], csems[fb])

            @pl.when(i + GA < CPW)
            def _():
                if (b + GA) % BCH == 0:
                    nmb = ((b + GA) % SLOTS) // BCH
                    pltpu.make_async_copy(
                        crw_hbm.at[wid, pl.ds((i + GA) // BCH * BCH, BCH)],
                        cbigs[nmb], csems[nmb]).wait()
                pltpu.async_copy(sup_hbm.at[cslice(b + GA).at[0]],
                                 rbufs[nrb], gsems[nrb])

            @pl.when(i < CPW)
            def _():
                pltpu.make_async_copy(sup_hbm.at[cslice(b).at[0]],
                                      rbufs[rb], gsems[rb]).wait()

                rbuf = rbufs[rb]
                cbuf = cslice(b)

                @pl.loop(0, CH // L)
                def _(g):
                    wvec = plsc.bitcast(cbuf[2, pl.ds(g * L, L)], jnp.float32)
                    for l in range(L):
                        sp = _splat(wvec, l)
                        e = g * L + l
                        for q in range(D // L):
                            sl = pl.ds(q * L, L)
                            rbuf[e, sl] = rbuf[e, sl] * sp

                pltpu.async_copy(rbuf, acc.at[cbuf.at[1]], ssems[rb],
                                 add=True)

    plsc.subcore_barrier()

    @pl.loop(0, ZPT)
    def _(k):
        @pl.when(k < ntr)
        def _():
            off = rbase + k * ZCH
            pltpu.sync_copy(acc.at[pl.ds(off, ZCH)],
                            out_hbm.at[cid, pl.ds(off, ZCH)])


@functools.cache
def _get_sc_agg():
    mesh = plsc.VectorSubcoreMesh(core_axis_name="c", subcore_axis_name="s",
                                  num_cores=NC, num_subcores=NS)
    cp = pltpu.CompilerParams()
    if "needs_layout_passes" in pltpu.CompilerParams.__dataclass_fields__:
        cp = dataclasses.replace(cp, needs_layout_passes=False)
    return pl.kernel(
        _sc_agg_body,
        out_type=jax.ShapeDtypeStruct((NC, N, D), jnp.float32),
        mesh=mesh,
        compiler_params=cp,
        scratch_types=[
            pltpu.VMEM_SHARED((N, D), jnp.float32),
            pltpu.VMEM((ZCH // 2, D), jnp.float32),
        ]
        + [pltpu.VMEM((BCH, 3, CH), jnp.int32) for _ in range(NBAT)]
        + [pltpu.VMEM((CH, D), jnp.float32) for _ in range(NB)]
        + [pltpu.SemaphoreType.DMA for _ in range(NBAT)]
        + [pltpu.SemaphoreType.DMA for _ in range(2 * NB)],
    )


BLK = 2000


def _mm1_body(x_ref, w_ref, o_ref):
    o_ref[...] = jnp.dot(x_ref[...], w_ref[...],
                         preferred_element_type=jnp.float32,
                         precision=lax.Precision.DEFAULT)


def _mid_body(p_ref, b_ref, w_ref, o_ref):
    h = p_ref[0] + p_ref[1] + b_ref[...]
    h = jnp.maximum(h, 0.0)
    o_ref[...] = jnp.dot(h, w_ref[...],
                         preferred_element_type=jnp.float32,
                         precision=lax.Precision.DEFAULT)


def _fin_body(p_ref, b_ref, o_ref):
    o_ref[...] = p_ref[0] + p_ref[1] + b_ref[...]


_mm1 = pl.pallas_call(
    _mm1_body,
    grid=(N // BLK,),
    in_specs=[
        pl.BlockSpec((BLK, D), lambda i: (i, 0)),
        pl.BlockSpec((D, D), lambda i: (0, 0)),
    ],
    out_specs=pl.BlockSpec((BLK, D), lambda i: (i, 0)),
    out_shape=jax.ShapeDtypeStruct((N, D), jnp.float32),
)

_mid = pl.pallas_call(
    _mid_body,
    grid=(N // BLK,),
    in_specs=[
        pl.BlockSpec((NC, BLK, D), lambda i: (0, i, 0)),
        pl.BlockSpec((1, D), lambda i: (0, 0)),
        pl.BlockSpec((D, D), lambda i: (0, 0)),
    ],
    out_specs=pl.BlockSpec((BLK, D), lambda i: (i, 0)),
    out_shape=jax.ShapeDtypeStruct((N, D), jnp.float32),
)

_fin = pl.pallas_call(
    _fin_body,
    grid=(N // BLK,),
    in_specs=[
        pl.BlockSpec((NC, BLK, D), lambda i: (0, i, 0)),
        pl.BlockSpec((1, D), lambda i: (0, 0)),
    ],
    out_specs=pl.BlockSpec((BLK, D), lambda i: (i, 0)),
    out_shape=jax.ShapeDtypeStruct((N, D), jnp.float32),
)


@jax.jit
def kernel(x, edge_index, edge_weight, W1, b1, W2, b2):
    ei = edge_index.astype(jnp.int32)
    col3 = ei[1].reshape(NW, CPW, 1, CH)
    row3 = ei[0].reshape(NW, CPW, 1, CH)
    wbits = lax.bitcast_convert_type(edge_weight, jnp.int32)
    w3 = wbits.reshape(NW, CPW, 1, CH)
    crw = jnp.concatenate([col3, row3, w3], axis=2)
    crw = jnp.pad(crw, ((0, 0), (0, CPW_PAD - CPW), (0, 0), (0, 0)))
    b1r = b1.reshape(1, D)
    b2r = b2.reshape(1, D)

    sc_agg = _get_sc_agg()
    s1 = _mm1(x, W1)
    p = sc_agg(s1, crw)
    s2 = _mid(p, b1r, W2)
    q = sc_agg(s2, crw)
    return _fin(q, b2r)
